# baseline (speedup 1.0000x reference)
import functools

import jax
import jax.numpy as jnp
from jax.experimental import pallas as pl

_Q, _K, _D, _S = 1024, 100000, 32, 50
_NUM_CLOSEST = 500
_TOPC = 552
_BK = 2048
_KPAD = 100352


def _sims_body(qn_ref, knt_ref, o_ref):
    i = pl.program_id(0)
    sims = jnp.dot(qn_ref[...], knt_ref[...], preferred_element_type=jnp.float32)
    col = i * _BK + jax.lax.broadcasted_iota(jnp.int32, sims.shape, 1)
    o_ref[...] = jnp.where(col < _K, sims, -jnp.inf)


def kernel(generated_embeddings, keys, seed_tracks):
    eps = 1e-8
    qn = generated_embeddings / jnp.maximum(
        jnp.linalg.norm(generated_embeddings, axis=-1, keepdims=True), eps)
    kn = keys / jnp.maximum(jnp.linalg.norm(keys, axis=-1, keepdims=True), eps)
    knt = jnp.pad(kn.T, ((0, 0), (0, _KPAD - _K)))

    sims = pl.pallas_call(
        _sims_body,
        grid=(_KPAD // _BK,),
        in_specs=[
            pl.BlockSpec((_Q, _D), lambda i: (0, 0)),
            pl.BlockSpec((_D, _BK), lambda i: (0, i)),
        ],
        out_specs=pl.BlockSpec((_Q, _BK), lambda i: (0, i)),
        out_shape=jax.ShapeDtypeStruct((_Q, _KPAD), jnp.float32),
    )(qn, knt)

    top_vals, top_idx = jax.lax.top_k(sims, _TOPC)
    in_seed = (top_idx[:, :, None] == seed_tracks[:, None, :]).any(axis=-1)
    order = jnp.argsort(in_seed, axis=1, stable=True)
    kept_idx = jnp.take_along_axis(top_idx, order, axis=1)[:, :_NUM_CLOSEST]
    kept_vals = jnp.take_along_axis(top_vals, order, axis=1)[:, :_NUM_CLOSEST]
    return kept_idx, kept_vals


# speedup vs baseline: 10.1808x; 10.1808x over previous
import functools

import jax
import jax.numpy as jnp
from jax import lax
from jax.experimental import pallas as pl
from jax.experimental.pallas import tpu as pltpu
from jax.experimental.pallas import tpu_sc as plsc

_Q, _K, _D, _S = 1024, 100000, 32, 50
_NUM_CLOSEST = 500
_TOPC = 552
_BK = 2048
_KPAD = 100352
_NBINS = 1024
_CAND = 2048
_NW = 32
_QPW = _Q // _NW
_NCHUNK = 8
_CHUNK = _KPAD // _NCHUNK
_VPC = _CHUNK // 16
_PAD_VAL = -3.0


def _sims_body(qn_ref, knt_ref, o_ref):
    i = pl.program_id(0)
    sims = jnp.dot(qn_ref[...], knt_ref[...], preferred_element_type=jnp.float32)
    col = i * _BK + lax.broadcasted_iota(jnp.int32, sims.shape, 1)
    o_ref[...] = jnp.where(col < _K, sims, _PAD_VAL)


def _bin_of(x):
    return jnp.clip(((x + 1.0) * 512.0).astype(jnp.int32), 0, _NBINS - 1)


def _select_body(sims_hbm, out_val, out_idx, chunk_buf, hist, cval, cidx):
    wid = lax.axis_index("s") * 2 + lax.axis_index("c")
    lane = lax.iota(jnp.int32, 16)
    ones = jnp.ones((16,), jnp.int32)
    zeros16 = jnp.zeros((16,), jnp.int32)
    fill16 = jnp.full((16,), _PAD_VAL, jnp.float32)

    def one_query(qi, _):
        q = wid * _QPW + qi

        def zero_hist(j, _):
            hist[pl.ds(j * 16, 16)] = zeros16
            return 0
        lax.fori_loop(0, 16 * _NBINS // 16, zero_hist, 0)

        def scan_a(c, _):
            pltpu.sync_copy(sims_hbm.at[q, pl.ds(c * _CHUNK, _CHUNK)], chunk_buf)

            def body(v, _):
                x = chunk_buf[pl.ds(v * 16, 16)]
                plsc.addupdate_scatter(hist, [lane * _NBINS + _bin_of(x)], ones)
                return 0
            lax.fori_loop(0, _VPC, body, 0)
            return 0
        lax.fori_loop(0, _NCHUNK, scan_a, 0)

        def thresh_block(j, carry):
            cum, nge = carry
            jj = _NBINS // 16 - 1 - j
            counts = zeros16
            for l in range(16):
                counts = counts + hist[pl.ds(l * _NBINS + jj * 16, 16)]
            suff = lax.rev(jnp.cumsum(lax.rev(counts, (0,))), (0,))
            nge = nge + jnp.sum(((suff + cum) >= _TOPC).astype(jnp.int32))
            cum = cum + jnp.sum(counts)
            return cum, nge
        _, nge = lax.fori_loop(0, _NBINS // 16, thresh_block,
                               (jnp.int32(0), jnp.int32(0)))
        bstar = nge - 1

        def fill(j, _):
            cval[pl.ds(j * 16, 16)] = fill16
            cidx[pl.ds(j * 16, 16)] = zeros16
            return 0
        lax.fori_loop(0, (_CAND + 16) // 16, fill, 0)

        def scan_b(c, cnt):
            pltpu.sync_copy(sims_hbm.at[q, pl.ds(c * _CHUNK, _CHUNK)], chunk_buf)

            def body(v, cnt):
                x = chunk_buf[pl.ds(v * 16, 16)]
                mask = _bin_of(x) >= bstar
                plsc.store_compressed(cval.at[pl.ds(cnt, 16)], x, mask=mask)
                gi = c * _CHUNK + v * 16 + lane
                plsc.store_compressed(cidx.at[pl.ds(cnt, 16)], gi, mask=mask)
                return jnp.minimum(cnt + jnp.sum(mask.astype(jnp.int32)),
                                   jnp.int32(_CAND))
            return lax.fori_loop(0, _VPC, body, cnt)
        lax.fori_loop(0, _NCHUNK, scan_b, jnp.int32(0))

        pltpu.sync_copy(cval.at[pl.ds(0, _CAND)], out_val.at[q])
        pltpu.sync_copy(cidx.at[pl.ds(0, _CAND)], out_idx.at[q])
        return 0

    lax.fori_loop(0, _QPW, one_query, 0)


_select = pl.kernel(
    _select_body,
    out_type=[jax.ShapeDtypeStruct((_Q, _CAND), jnp.float32),
              jax.ShapeDtypeStruct((_Q, _CAND), jnp.int32)],
    mesh=plsc.VectorSubcoreMesh(core_axis_name="c", subcore_axis_name="s"),
    compiler_params=pltpu.CompilerParams(needs_layout_passes=False),
    scratch_types=[
        pltpu.VMEM((_CHUNK,), jnp.float32),
        pltpu.VMEM((16 * _NBINS,), jnp.int32),
        pltpu.VMEM((_CAND + 16,), jnp.float32),
        pltpu.VMEM((_CAND + 16,), jnp.int32),
    ],
)


def kernel(generated_embeddings, keys, seed_tracks):
    eps = 1e-8
    qn = generated_embeddings / jnp.maximum(
        jnp.linalg.norm(generated_embeddings, axis=-1, keepdims=True), eps)
    kn = keys / jnp.maximum(jnp.linalg.norm(keys, axis=-1, keepdims=True), eps)
    knt = jnp.pad(kn.T, ((0, 0), (0, _KPAD - _K)))

    sims = pl.pallas_call(
        _sims_body,
        grid=(_KPAD // _BK,),
        in_specs=[
            pl.BlockSpec((_Q, _D), lambda i: (0, 0)),
            pl.BlockSpec((_D, _BK), lambda i: (0, i)),
        ],
        out_specs=pl.BlockSpec((_Q, _BK), lambda i: (0, i)),
        out_shape=jax.ShapeDtypeStruct((_Q, _KPAD), jnp.float32),
    )(qn, knt)

    cand_val, cand_idx = _select(sims)

    top_vals, pos = lax.top_k(cand_val, _TOPC)
    top_idx = jnp.take_along_axis(cand_idx, pos, axis=1)
    in_seed = (top_idx[:, :, None] == seed_tracks[:, None, :]).any(axis=-1)
    order = jnp.argsort(in_seed, axis=1, stable=True)
    kept_idx = jnp.take_along_axis(top_idx, order, axis=1)[:, :_NUM_CLOSEST]
    kept_vals = jnp.take_along_axis(top_vals, order, axis=1)[:, :_NUM_CLOSEST]
    return kept_idx, kept_vals


# speedup vs baseline: 10.2537x; 1.0072x over previous
import functools

import jax
import jax.numpy as jnp
from jax import lax
from jax.experimental import pallas as pl
from jax.experimental.pallas import tpu as pltpu
from jax.experimental.pallas import tpu_sc as plsc

_Q, _K, _D, _S = 1024, 100000, 32, 50
_NUM_CLOSEST = 500
_TOPC = 552
_BK = 2048
_KPAD = 100352
_NBINS = 1024
_HSTRIDE = 1041
_CAND = 2048
_NW = 32
_QPW = _Q // _NW
_NCHUNK = 8
_CHUNK = _KPAD // _NCHUNK
_VPR = _KPAD // 16
_PAD_VAL = -3.0
_PAD_IDX = _K


def _sims_body(qn_ref, knt_ref, o_ref):
    i = pl.program_id(0)
    sims = jnp.dot(qn_ref[...], knt_ref[...], preferred_element_type=jnp.float32)
    col = i * _BK + lax.broadcasted_iota(jnp.int32, sims.shape, 1)
    o_ref[...] = jnp.where(col < _K, sims, _PAD_VAL)


def _select_body(sims_hbm, out_idx, row, hist, cidx, sem):
    wid = lax.axis_index("s") * 2 + lax.axis_index("c")
    lane = lax.iota(jnp.int32, 16)
    lane_f = lane.astype(jnp.float32)
    a_off = 512.0 + lane_f * float(_HSTRIDE)
    lo_vec = lane * _HSTRIDE
    hi_vec = lo_vec + (_NBINS - 1)
    ones = jnp.ones((16,), jnp.int32)
    zeros16 = jnp.zeros((16,), jnp.int32)
    pad_idx16 = jnp.full((16,), _PAD_IDX, jnp.int32)
    cap16 = jnp.full((16,), _CAND + 15, jnp.int32)

    def one_query(qi, _):
        q = wid * _QPW + qi

        copies = [
            pltpu.async_copy(
                sims_hbm.at[q, pl.ds(c * _CHUNK, _CHUNK)],
                row.at[pl.ds(c * _CHUNK, _CHUNK)], sem)
            for c in range(_NCHUNK)
        ]

        def zero_hist(j, _):
            hist[pl.ds(j * 16, 16)] = zeros16
            return 0
        lax.fori_loop(0, _HSTRIDE, zero_hist, 0, unroll=4)

        vpc = _CHUNK // 16
        for c in range(_NCHUNK):
            copies[c].wait()

            def body_a(v, _):
                x = row[pl.ds(c * _CHUNK + v * 16, 16)]
                b = ((x * 512.0) + a_off).astype(jnp.int32)
                b = jnp.minimum(jnp.maximum(b, lo_vec), hi_vec)
                plsc.addupdate_scatter(hist, [b], ones)
                return 0
            lax.fori_loop(0, vpc, body_a, 0, unroll=8)

        def thresh_block(j, carry):
            cum, nge = carry
            jj = _NBINS // 16 - 1 - j
            counts = zeros16
            for l in range(16):
                counts = counts + hist[pl.ds(l * _HSTRIDE + jj * 16, 16)]
            suff = lax.rev(jnp.cumsum(lax.rev(counts, (0,))), (0,))
            nge = nge + jnp.sum(((suff + cum) >= _TOPC).astype(jnp.int32))
            cum = cum + jnp.sum(counts)
            return cum, nge
        _, nge = lax.fori_loop(0, _NBINS // 16, thresh_block,
                               (jnp.int32(0), jnp.int32(0)))
        bstar = nge - 1

        def fill(j, _):
            cidx[pl.ds(j * 16, 16)] = pad_idx16
            return 0
        lax.fori_loop(0, (_CAND + 16) // 16, fill, 0, unroll=4)

        def body_b(v, carry):
            cntm1, giv = carry
            x = row[pl.ds(v * 16, 16)]
            b = ((x * 512.0) + 512.0).astype(jnp.int32)
            mask = b >= bstar
            rank = plsc.cumsum(ones, mask=mask)
            pos = jnp.minimum(cntm1 + rank, cap16)
            plsc.store_scatter(cidx, [pos], giv, mask=mask)
            pc = plsc.all_reduce_population_count(mask)
            return cntm1 + pc, giv + 16
        lax.fori_loop(0, _VPR, body_b,
                      (jnp.full((16,), -1, jnp.int32), lane), unroll=8)

        pltpu.sync_copy(cidx.at[pl.ds(0, _CAND)], out_idx.at[q])
        return 0

    lax.fori_loop(0, _QPW, one_query, 0)


_select = pl.kernel(
    _select_body,
    out_type=jax.ShapeDtypeStruct((_Q, _CAND), jnp.int32),
    mesh=plsc.VectorSubcoreMesh(core_axis_name="c", subcore_axis_name="s"),
    compiler_params=pltpu.CompilerParams(needs_layout_passes=False),
    scratch_types=[
        pltpu.VMEM((_KPAD,), jnp.float32),
        pltpu.VMEM((16 * _HSTRIDE,), jnp.int32),
        pltpu.VMEM((_CAND + 16,), jnp.int32),
        pltpu.SemaphoreType.DMA,
    ],
)


def kernel(generated_embeddings, keys, seed_tracks):
    eps = 1e-8
    qn = generated_embeddings / jnp.maximum(
        jnp.linalg.norm(generated_embeddings, axis=-1, keepdims=True), eps)
    kn = keys / jnp.maximum(jnp.linalg.norm(keys, axis=-1, keepdims=True), eps)
    knt = jnp.pad(kn.T, ((0, 0), (0, _KPAD - _K)))

    sims = pl.pallas_call(
        _sims_body,
        grid=(_KPAD // _BK,),
        in_specs=[
            pl.BlockSpec((_Q, _D), lambda i: (0, 0)),
            pl.BlockSpec((_D, _BK), lambda i: (0, i)),
        ],
        out_specs=pl.BlockSpec((_Q, _BK), lambda i: (0, i)),
        out_shape=jax.ShapeDtypeStruct((_Q, _KPAD), jnp.float32),
    )(qn, knt)

    cand_idx = _select(sims)
    cand_val = jnp.take_along_axis(sims, cand_idx, axis=1)

    top_vals, pos = lax.top_k(cand_val, _TOPC)
    top_idx = jnp.take_along_axis(cand_idx, pos, axis=1)
    in_seed = (top_idx[:, :, None] == seed_tracks[:, None, :]).any(axis=-1)
    order = jnp.argsort(in_seed, axis=1, stable=True)
    kept_idx = jnp.take_along_axis(top_idx, order, axis=1)[:, :_NUM_CLOSEST]
    kept_vals = jnp.take_along_axis(top_vals, order, axis=1)[:, :_NUM_CLOSEST]
    return kept_idx, kept_vals


# speedup vs baseline: 36.4678x; 3.5566x over previous
import functools

import jax
import jax.numpy as jnp
from jax import lax
from jax.experimental import pallas as pl
from jax.experimental.pallas import tpu as pltpu
from jax.experimental.pallas import tpu_sc as plsc

_Q, _K, _D, _S = 1024, 100000, 32, 50
_NUM_CLOSEST = 500
_TOPC = 552
_BK = 2048
_KPAD = 100352
_NBINS = 1024
_HSTRIDE = 1041
_CAND = 2048
_NW = 32
_QPW = _Q // _NW
_NCHUNK = 8
_CHUNK = _KPAD // _NCHUNK
_VPR = _KPAD // 16
_PAD_VAL = -3.0
_PAD_IDX = _K


def _sims_body(qn_ref, knt_ref, o_ref):
    i = pl.program_id(0)
    sims = jnp.dot(qn_ref[...], knt_ref[...], preferred_element_type=jnp.float32)
    col = i * _BK + lax.broadcasted_iota(jnp.int32, sims.shape, 1)
    o_ref[...] = jnp.where(col < _K, sims, _PAD_VAL)


def _select_body(sims_hbm, out_idx, row, hist, cidx, sem):
    wid = lax.axis_index("s") * 2 + lax.axis_index("c")
    lane = lax.iota(jnp.int32, 16)
    lane_f = lane.astype(jnp.float32)
    a_off = 512.0 + lane_f * float(_HSTRIDE)
    lo_vec = lane * _HSTRIDE
    hi_vec = lo_vec + (_NBINS - 1)
    ones = jnp.ones((16,), jnp.int32)
    zeros16 = jnp.zeros((16,), jnp.int32)
    pad_idx16 = jnp.full((16,), _PAD_IDX, jnp.int32)
    cap16 = jnp.full((16,), _CAND + 15, jnp.int32)

    def one_query(qi, _):
        q = wid * _QPW + qi

        copies = [
            pltpu.async_copy(
                sims_hbm.at[q, pl.ds(c * _CHUNK, _CHUNK)],
                row.at[pl.ds(c * _CHUNK, _CHUNK)], sem)
            for c in range(_NCHUNK)
        ]

        def zero_hist(j, _):
            hist[pl.ds(j * 16, 16)] = zeros16
            return 0
        lax.fori_loop(0, _HSTRIDE, zero_hist, 0, unroll=4)

        vpc = _CHUNK // 16
        for c in range(_NCHUNK):
            copies[c].wait()

            @plsc.parallel_loop(0, vpc, unroll=8)
            def body_a(v):
                x = row[pl.ds(c * _CHUNK + v * 16, 16)]
                b = ((x * 512.0) + a_off).astype(jnp.int32)
                b = jnp.minimum(jnp.maximum(b, lo_vec), hi_vec)
                plsc.addupdate_scatter(hist, [b], ones)

        def thresh_block(j, carry):
            cum, nge = carry
            jj = _NBINS // 16 - 1 - j
            counts = zeros16
            for l in range(16):
                counts = counts + hist[pl.ds(l * _HSTRIDE + jj * 16, 16)]
            suff = lax.rev(jnp.cumsum(lax.rev(counts, (0,))), (0,))
            nge = nge + jnp.sum(((suff + cum) >= _TOPC).astype(jnp.int32))
            cum = cum + jnp.sum(counts)
            return cum, nge
        _, nge = lax.fori_loop(0, _NBINS // 16, thresh_block,
                               (jnp.int32(0), jnp.int32(0)))
        bstar = nge - 1

        def fill(j, _):
            cidx[pl.ds(j * 16, 16)] = pad_idx16
            return 0
        lax.fori_loop(0, (_CAND + 16) // 16, fill, 0, unroll=4)

        @plsc.parallel_loop(0, _VPR, unroll=8,
                            carry=(jnp.full((16,), -1, jnp.int32), lane))
        def body_b(v, carry):
            cntm1, giv = carry
            x = row[pl.ds(v * 16, 16)]
            b = ((x * 512.0) + 512.0).astype(jnp.int32)
            mask = b >= bstar
            rank = plsc.cumsum(ones, mask=mask)
            pos = jnp.minimum(cntm1 + rank, cap16)
            plsc.store_scatter(cidx, [pos], giv, mask=mask)
            pc = plsc.all_reduce_population_count(mask)
            return cntm1 + pc, giv + 16

        pltpu.sync_copy(cidx.at[pl.ds(0, _CAND)], out_idx.at[q])
        return 0

    lax.fori_loop(0, _QPW, one_query, 0)


_select = pl.kernel(
    _select_body,
    out_type=jax.ShapeDtypeStruct((_Q, _CAND), jnp.int32),
    mesh=plsc.VectorSubcoreMesh(core_axis_name="c", subcore_axis_name="s"),
    compiler_params=pltpu.CompilerParams(needs_layout_passes=False),
    scratch_types=[
        pltpu.VMEM((_KPAD,), jnp.float32),
        pltpu.VMEM((16 * _HSTRIDE,), jnp.int32),
        pltpu.VMEM((_CAND + 16,), jnp.int32),
        pltpu.SemaphoreType.DMA,
    ],
)


def kernel(generated_embeddings, keys, seed_tracks):
    eps = 1e-8
    qn = generated_embeddings / jnp.maximum(
        jnp.linalg.norm(generated_embeddings, axis=-1, keepdims=True), eps)
    kn = keys / jnp.maximum(jnp.linalg.norm(keys, axis=-1, keepdims=True), eps)
    knt = jnp.pad(kn.T, ((0, 0), (0, _KPAD - _K)))

    sims = pl.pallas_call(
        _sims_body,
        grid=(_KPAD // _BK,),
        in_specs=[
            pl.BlockSpec((_Q, _D), lambda i: (0, 0)),
            pl.BlockSpec((_D, _BK), lambda i: (0, i)),
        ],
        out_specs=pl.BlockSpec((_Q, _BK), lambda i: (0, i)),
        out_shape=jax.ShapeDtypeStruct((_Q, _KPAD), jnp.float32),
    )(qn, knt)

    cand_idx = _select(sims)
    cand_val = jnp.take_along_axis(sims, cand_idx, axis=1)

    top_vals, pos = lax.top_k(cand_val, _TOPC)
    top_idx = jnp.take_along_axis(cand_idx, pos, axis=1)
    in_seed = (top_idx[:, :, None] == seed_tracks[:, None, :]).any(axis=-1)
    order = jnp.argsort(in_seed, axis=1, stable=True)
    kept_idx = jnp.take_along_axis(top_idx, order, axis=1)[:, :_NUM_CLOSEST]
    kept_vals = jnp.take_along_axis(top_vals, order, axis=1)[:, :_NUM_CLOSEST]
    return kept_idx, kept_vals


# speedup vs baseline: 50.7029x; 1.3903x over previous
import functools

import jax
import jax.numpy as jnp
from jax import lax
from jax.experimental import pallas as pl
from jax.experimental.pallas import tpu as pltpu
from jax.experimental.pallas import tpu_sc as plsc

_Q, _K, _D, _S = 1024, 100000, 32, 50
_NUM_CLOSEST = 500
_TOPC = 552
_BK = 2048
_KPAD = 100352
_NBINS = 1024
_HSTRIDE = 1041
_CAND = 1024
_NW = 32
_QPW = _Q // _NW
_NCHUNK = 8
_CHUNK = _KPAD // _NCHUNK
_VPR = _KPAD // 16
_PAD_VAL = -3.0
_PAD_IDX = _K


def _sims_body(qn_ref, knt_ref, o_ref):
    i = pl.program_id(0)
    sims = jnp.dot(qn_ref[...], knt_ref[...], preferred_element_type=jnp.float32)
    col = i * _BK + lax.broadcasted_iota(jnp.int32, sims.shape, 1)
    o_ref[...] = jnp.where(col < _K, sims, _PAD_VAL)


def _select_body(sims_hbm, out_idx, out_val, row, hist, cidx, cval, sem):
    wid = lax.axis_index("s") * 2 + lax.axis_index("c")
    lane = lax.iota(jnp.int32, 16)
    lane_f = lane.astype(jnp.float32)
    a_off = 512.0 + lane_f * float(_HSTRIDE)
    lo_vec = lane * _HSTRIDE
    hi_vec = lo_vec + (_NBINS - 1)
    ones = jnp.ones((16,), jnp.int32)
    zeros16 = jnp.zeros((16,), jnp.int32)
    pad_idx16 = jnp.full((16,), _PAD_IDX, jnp.int32)
    cap16 = jnp.full((16,), _CAND + 15, jnp.int32)

    def one_query(qi, _):
        q = wid * _QPW + qi

        copies = [
            pltpu.async_copy(
                sims_hbm.at[q, pl.ds(c * _CHUNK, _CHUNK)],
                row.at[pl.ds(c * _CHUNK, _CHUNK)], sem)
            for c in range(_NCHUNK)
        ]

        def zero_hist(j, _):
            hist[pl.ds(j * 16, 16)] = zeros16
            return 0
        lax.fori_loop(0, _HSTRIDE, zero_hist, 0, unroll=4)

        vpc = _CHUNK // 16
        for c in range(_NCHUNK):
            copies[c].wait()

            @plsc.parallel_loop(0, vpc, unroll=8)
            def body_a(v):
                x = row[pl.ds(c * _CHUNK + v * 16, 16)]
                b = ((x * 512.0) + a_off).astype(jnp.int32)
                b = jnp.minimum(jnp.maximum(b, lo_vec), hi_vec)
                plsc.addupdate_scatter(hist, [b], ones)

        def thresh_block(j, carry):
            cum, nge = carry
            jj = _NBINS // 16 - 1 - j
            counts = zeros16
            for l in range(16):
                counts = counts + hist[pl.ds(l * _HSTRIDE + jj * 16, 16)]
            suff = lax.rev(jnp.cumsum(lax.rev(counts, (0,))), (0,))
            nge = nge + jnp.sum(((suff + cum) >= _TOPC).astype(jnp.int32))
            cum = cum + jnp.sum(counts)
            return cum, nge
        _, nge = lax.fori_loop(0, _NBINS // 16, thresh_block,
                               (jnp.int32(0), jnp.int32(0)))
        bstar = nge - 1

        pad_val16 = jnp.full((16,), _PAD_VAL, jnp.float32)

        def fill(j, _):
            cidx[pl.ds(j * 16, 16)] = pad_idx16
            cval[pl.ds(j * 16, 16)] = pad_val16
            return 0
        lax.fori_loop(0, (_CAND + 16) // 16, fill, 0, unroll=4)

        @plsc.parallel_loop(0, _VPR, unroll=8,
                            carry=(jnp.full((16,), -1, jnp.int32), lane))
        def body_b(v, carry):
            cntm1, giv = carry
            x = row[pl.ds(v * 16, 16)]
            b = ((x * 512.0) + 512.0).astype(jnp.int32)
            mask = b >= bstar
            rank = plsc.cumsum(ones, mask=mask)
            pos = jnp.minimum(cntm1 + rank, cap16)
            plsc.store_scatter(cidx, [pos], giv, mask=mask)
            plsc.store_scatter(cval, [pos], x, mask=mask)
            pc = plsc.all_reduce_population_count(mask)
            return cntm1 + pc, giv + 16

        pltpu.sync_copy(cidx.at[pl.ds(0, _CAND)], out_idx.at[q])
        pltpu.sync_copy(cval.at[pl.ds(0, _CAND)], out_val.at[q])
        return 0

    lax.fori_loop(0, _QPW, one_query, 0)


_select = pl.kernel(
    _select_body,
    out_type=[jax.ShapeDtypeStruct((_Q, _CAND), jnp.int32),
              jax.ShapeDtypeStruct((_Q, _CAND), jnp.float32)],
    mesh=plsc.VectorSubcoreMesh(core_axis_name="c", subcore_axis_name="s"),
    compiler_params=pltpu.CompilerParams(needs_layout_passes=False),
    scratch_types=[
        pltpu.VMEM((_KPAD,), jnp.float32),
        pltpu.VMEM((16 * _HSTRIDE,), jnp.int32),
        pltpu.VMEM((_CAND + 16,), jnp.int32),
        pltpu.VMEM((_CAND + 16,), jnp.float32),
        pltpu.SemaphoreType.DMA,
    ],
)


def kernel(generated_embeddings, keys, seed_tracks):
    eps = 1e-8
    qn = generated_embeddings / jnp.maximum(
        jnp.linalg.norm(generated_embeddings, axis=-1, keepdims=True), eps)
    kn = keys / jnp.maximum(jnp.linalg.norm(keys, axis=-1, keepdims=True), eps)
    knt = jnp.pad(kn.T, ((0, 0), (0, _KPAD - _K)))

    sims = pl.pallas_call(
        _sims_body,
        grid=(_KPAD // _BK,),
        in_specs=[
            pl.BlockSpec((_Q, _D), lambda i: (0, 0)),
            pl.BlockSpec((_D, _BK), lambda i: (0, i)),
        ],
        out_specs=pl.BlockSpec((_Q, _BK), lambda i: (0, i)),
        out_shape=jax.ShapeDtypeStruct((_Q, _KPAD), jnp.float32),
    )(qn, knt)

    cand_idx, cand_val = _select(sims)

    top_vals, pos = lax.top_k(cand_val, _TOPC)
    top_idx = jnp.take_along_axis(cand_idx, pos, axis=1)
    in_seed = (top_idx[:, :, None] == seed_tracks[:, None, :]).any(axis=-1)
    order = jnp.argsort(in_seed, axis=1, stable=True)
    kept_idx = jnp.take_along_axis(top_idx, order, axis=1)[:, :_NUM_CLOSEST]
    kept_vals = jnp.take_along_axis(top_vals, order, axis=1)[:, :_NUM_CLOSEST]
    return kept_idx, kept_vals


# speedup vs baseline: 50.7198x; 1.0003x over previous
import functools

import jax
import jax.numpy as jnp
from jax import lax
from jax.experimental import pallas as pl
from jax.experimental.pallas import tpu as pltpu
from jax.experimental.pallas import tpu_sc as plsc

_Q, _K, _D, _S = 1024, 100000, 32, 50
_NUM_CLOSEST = 500
_TOPC = 552
_BK = 2048
_KPAD = 100352
_NBINS = 1024
_HSTRIDE = 1041
_CAND = 1024
_NW = 32
_QPW = _Q // _NW
_NCHUNK = 8
_CHUNK = _KPAD // _NCHUNK
_VPR = _KPAD // 16
_PAD_VAL = -3.0
_PAD_IDX = _K
_SPAD = 64
_KOUT = 512
_QB = 128


def _sims_body(qn_ref, knt_ref, o_ref):
    i = pl.program_id(0)
    sims = jnp.dot(qn_ref[...], knt_ref[...], preferred_element_type=jnp.float32)
    col = i * _BK + lax.broadcasted_iota(jnp.int32, sims.shape, 1)
    o_ref[...] = jnp.where(col < _K, sims, _PAD_VAL)


def _select_body(sims_hbm, out_idx, out_val, row, hist, cidx, cval, sem):
    wid = lax.axis_index("s") * 2 + lax.axis_index("c")
    lane = lax.iota(jnp.int32, 16)
    lane_f = lane.astype(jnp.float32)
    a_off = 512.0 + lane_f * float(_HSTRIDE)
    lo_vec = lane * _HSTRIDE
    hi_vec = lo_vec + (_NBINS - 1)
    ones = jnp.ones((16,), jnp.int32)
    zeros16 = jnp.zeros((16,), jnp.int32)
    pad_idx16 = jnp.full((16,), _PAD_IDX, jnp.int32)
    cap16 = jnp.full((16,), _CAND + 15, jnp.int32)

    def one_query(qi, _):
        q = wid * _QPW + qi

        copies = [
            pltpu.async_copy(
                sims_hbm.at[q, pl.ds(c * _CHUNK, _CHUNK)],
                row.at[pl.ds(c * _CHUNK, _CHUNK)], sem)
            for c in range(_NCHUNK)
        ]

        def zero_hist(j, _):
            hist[pl.ds(j * 16, 16)] = zeros16
            return 0
        lax.fori_loop(0, _HSTRIDE, zero_hist, 0, unroll=4)

        vpc = _CHUNK // 16
        for c in range(_NCHUNK):
            copies[c].wait()

            @plsc.parallel_loop(0, vpc, unroll=8)
            def body_a(v):
                x = row[pl.ds(c * _CHUNK + v * 16, 16)]
                b = ((x * 512.0) + a_off).astype(jnp.int32)
                b = jnp.minimum(jnp.maximum(b, lo_vec), hi_vec)
                plsc.addupdate_scatter(hist, [b], ones)

        def thresh_block(j, carry):
            cum, nge = carry
            jj = _NBINS // 16 - 1 - j
            counts = zeros16
            for l in range(16):
                counts = counts + hist[pl.ds(l * _HSTRIDE + jj * 16, 16)]
            suff = lax.rev(jnp.cumsum(lax.rev(counts, (0,))), (0,))
            nge = nge + jnp.sum(((suff + cum) >= _TOPC).astype(jnp.int32))
            cum = cum + jnp.sum(counts)
            return cum, nge
        _, nge = lax.fori_loop(0, _NBINS // 16, thresh_block,
                               (jnp.int32(0), jnp.int32(0)))
        bstar = nge - 1

        pad_val16 = jnp.full((16,), _PAD_VAL, jnp.float32)

        def fill(j, _):
            cidx[pl.ds(j * 16, 16)] = pad_idx16
            cval[pl.ds(j * 16, 16)] = pad_val16
            return 0
        lax.fori_loop(0, (_CAND + 16) // 16, fill, 0, unroll=4)

        @plsc.parallel_loop(0, _VPR, unroll=8,
                            carry=(jnp.full((16,), -1, jnp.int32), lane))
        def body_b(v, carry):
            cntm1, giv = carry
            x = row[pl.ds(v * 16, 16)]
            b = ((x * 512.0) + 512.0).astype(jnp.int32)
            mask = b >= bstar
            rank = plsc.cumsum(ones, mask=mask)
            pos = jnp.minimum(cntm1 + rank, cap16)
            plsc.store_scatter(cidx, [pos], giv, mask=mask)
            plsc.store_scatter(cval, [pos], x, mask=mask)
            pc = plsc.all_reduce_population_count(mask)
            return cntm1 + pc, giv + 16

        pltpu.sync_copy(cidx.at[pl.ds(0, _CAND)], out_idx.at[q])
        pltpu.sync_copy(cval.at[pl.ds(0, _CAND)], out_val.at[q])
        return 0

    lax.fori_loop(0, _QPW, one_query, 0)


@functools.cache
def _make_select():
    return pl.kernel(
        _select_body,
        out_type=[jax.ShapeDtypeStruct((_Q, _CAND), jnp.int32),
                  jax.ShapeDtypeStruct((_Q, _CAND), jnp.float32)],
        mesh=plsc.VectorSubcoreMesh(core_axis_name="c", subcore_axis_name="s"),
        compiler_params=pltpu.CompilerParams(needs_layout_passes=False),
        scratch_types=[
            pltpu.VMEM((_KPAD,), jnp.float32),
            pltpu.VMEM((16 * _HSTRIDE,), jnp.int32),
            pltpu.VMEM((_CAND + 16,), jnp.int32),
            pltpu.VMEM((_CAND + 16,), jnp.float32),
            pltpu.SemaphoreType.DMA,
        ],
    )


def kernel(generated_embeddings, keys, seed_tracks):
    eps = 1e-8
    qn = generated_embeddings / jnp.maximum(
        jnp.linalg.norm(generated_embeddings, axis=-1, keepdims=True), eps)
    kn = keys / jnp.maximum(jnp.linalg.norm(keys, axis=-1, keepdims=True), eps)
    knt = jnp.pad(kn.T, ((0, 0), (0, _KPAD - _K)))

    sims = pl.pallas_call(
        _sims_body,
        grid=(_KPAD // _BK,),
        in_specs=[
            pl.BlockSpec((_Q, _D), lambda i: (0, 0)),
            pl.BlockSpec((_D, _BK), lambda i: (0, i)),
        ],
        out_specs=pl.BlockSpec((_Q, _BK), lambda i: (0, i)),
        out_shape=jax.ShapeDtypeStruct((_Q, _KPAD), jnp.float32),
    )(qn, knt)

    cand_idx, cand_val = _make_select()(sims)

    top_vals, pos = lax.top_k(cand_val, _TOPC)
    top_idx = jnp.take_along_axis(cand_idx, pos, axis=1)
    in_seed = (top_idx[:, :, None] == seed_tracks[:, None, :]).any(axis=-1)
    order = jnp.argsort(in_seed, axis=1, stable=True)
    kept_idx = jnp.take_along_axis(top_idx, order, axis=1)[:, :_NUM_CLOSEST]
    kept_vals = jnp.take_along_axis(top_vals, order, axis=1)[:, :_NUM_CLOSEST]
    return kept_idx, kept_vals
